# baseline (speedup 1.0000x reference)
import functools

import jax
import jax.numpy as jnp
from jax import lax
from jax.experimental import pallas as pl
from jax.experimental.pallas import tpu as pltpu
from jax.experimental.pallas import tpu_sc as plsc

_ED = 64
_NB = 8
_NBLK = _NB * _NB * _NB
_BATCH = 1024
_NW = 32
_BPW = _NBLK // _NW
_GW = 128
_NCH = _BATCH // _GW
_TPAD = 65
_TROWS = 1008


def _sc_gather(table65, gi_tiles):
    mesh = plsc.VectorSubcoreMesh(core_axis_name="c", subcore_axis_name="s")

    @functools.partial(
        pl.kernel,
        mesh=mesh,
        out_type=jax.ShapeDtypeStruct((4096, 8, 8, 128), jnp.float32),
        scratch_types=[
            pltpu.VMEM((_TROWS * _TPAD,), jnp.float32),
            pltpu.VMEM((_BPW, _NCH, _GW), jnp.int32),
            [pltpu.VMEM((8, 8, 128), jnp.float32) for _ in range(2)],
            [pltpu.SemaphoreType.DMA for _ in range(2)],
            pltpu.SemaphoreType.DMA,
            pltpu.SemaphoreType.DMA,
        ],
        compiler_params=pltpu.CompilerParams(
            use_tc_tiling_on_sc=False,
            needs_layout_passes=False,
            disable_bounds_checks=True,
        ),
    )
    def k(tbl_hbm, gi_hbm, out_hbm, tbl, idx_v, stage, wsem, isem, tsem):
        wid = lax.axis_index("s") * 2 + lax.axis_index("c")
        pltpu.async_copy(gi_hbm.at[wid], idx_v, isem)
        pltpu.async_copy(tbl_hbm, tbl, tsem)
        pltpu.make_async_copy(gi_hbm.at[wid], idx_v, isem).wait()
        pltpu.make_async_copy(tbl_hbm, tbl, tsem).wait()

        def write_copy(blk, g, sb):
            base = pl.multiple_of((wid * _BPW + blk) * 8, 8)
            return pltpu.make_async_copy(
                stage[sb], out_hbm.at[pl.ds(base, 8), g], wsem[sb])

        def block_body(blk, carry):
            def g2_body(g2, carry2):
                for gp in range(2):
                    g = g2 * 2 + gp
                    sb = gp

                    @pl.when(blk * _NCH + g >= 2)
                    def _():
                        write_copy(blk, g, sb).wait()

                    for lgrp in range(8):
                        iv = idx_v[blk, g, pl.ds(lgrp * 16, 16)]
                        base = iv * _TPAD
                        for e in range(_ED):
                            v = plsc.load_gather(tbl, [base + e])
                            stage[sb][e >> 3, e & 7, pl.ds(lgrp * 16, 16)] = v

                    write_copy(blk, g, sb).start()
                return carry2

            lax.fori_loop(0, _NCH // 2, g2_body, 0)
            return carry

        lax.fori_loop(0, _BPW, block_body, 0)
        for sb in range(2):
            write_copy(_BPW - 1, 6 + sb, sb).wait()

    return k(table65, gi_tiles)


def kernel(x, gi, ee):
    table = jnp.concatenate([ee, x], axis=0)
    table65 = jnp.pad(table, ((0, _TROWS - table.shape[0]), (0, 1)))
    git = gi.astype(jnp.int32).T.reshape(_NW, _BPW, _NCH, _GW)
    buf = _sc_gather(table65.reshape(-1), git)
    r = buf.reshape(_NBLK, 8, 8, 8, 128)
    out = r.transpose(2, 4, 0, 1, 3)
    return out.reshape(gi.shape[0], _NB, _NB, _NB, _ED)


# speedup vs baseline: 5.0348x; 5.0348x over previous
import functools

import jax
import jax.numpy as jnp
from jax import lax
from jax.experimental import pallas as pl
from jax.experimental.pallas import tpu as pltpu
from jax.experimental.pallas import tpu_sc as plsc

_ED = 64
_NB = 8
_NBLK = _NB * _NB * _NB
_BATCH = 1024
_NW = 32
_BPW = _NBLK // _NW
_GW = 128
_NCH = _BATCH // _GW
_TPAD = 65
_TROWS = 1008


def _sc_gather(table65, gi_tiles):
    mesh = plsc.VectorSubcoreMesh(core_axis_name="c", subcore_axis_name="s")

    @functools.partial(
        pl.kernel,
        mesh=mesh,
        out_type=jax.ShapeDtypeStruct((4096, 8, 8, 128), jnp.float32),
        scratch_types=[
            pltpu.VMEM((_TROWS * _TPAD,), jnp.float32),
            pltpu.VMEM((_BPW, _NCH, _GW), jnp.int32),
            [pltpu.VMEM((8, 8, 128), jnp.float32) for _ in range(2)],
            [pltpu.SemaphoreType.DMA for _ in range(2)],
            pltpu.SemaphoreType.DMA,
            pltpu.SemaphoreType.DMA,
        ],
        compiler_params=pltpu.CompilerParams(
            use_tc_tiling_on_sc=False,
            needs_layout_passes=False,
            disable_bounds_checks=True,
        ),
    )
    def k(tbl_hbm, gi_hbm, out_hbm, tbl, idx_v, stage, wsem, isem, tsem):
        wid = lax.axis_index("s") * 2 + lax.axis_index("c")
        pltpu.async_copy(gi_hbm.at[wid], idx_v, isem)
        pltpu.async_copy(tbl_hbm, tbl, tsem)
        pltpu.make_async_copy(gi_hbm.at[wid], idx_v, isem).wait()
        pltpu.make_async_copy(tbl_hbm, tbl, tsem).wait()

        def write_copy(blk, g, sb):
            base = pl.multiple_of((wid * _BPW + blk) * 8, 8)
            return pltpu.make_async_copy(
                stage[sb], out_hbm.at[pl.ds(base, 8), g], wsem[sb])

        def block_body(blk, carry):
            def g2_body(g2, carry2):
                for gp in range(2):
                    g = g2 * 2 + gp
                    sb = gp

                    @pl.when(blk * _NCH + g >= 2)
                    def _():
                        write_copy(blk, g, sb).wait()

                    for lgrp in range(8):
                        iv = idx_v[blk, g, pl.ds(lgrp * 16, 16)]
                        base = iv * _TPAD

                        @plsc.parallel_loop(0, _ED, step=1, unroll=16)
                        def _(e):
                            v = plsc.load_gather(tbl, [base + e])
                            stage[sb][e >> 3, e & 7, pl.ds(lgrp * 16, 16)] = v

                    write_copy(blk, g, sb).start()
                return carry2

            lax.fori_loop(0, _NCH // 2, g2_body, 0)
            return carry

        lax.fori_loop(0, _BPW, block_body, 0)
        for sb in range(2):
            write_copy(_BPW - 1, 6 + sb, sb).wait()

    return k(table65, gi_tiles)


def kernel(x, gi, ee):
    table = jnp.concatenate([ee, x], axis=0)
    table65 = jnp.pad(table, ((0, _TROWS - table.shape[0]), (0, 1)))
    git = gi.astype(jnp.int32).T.reshape(_NW, _BPW, _NCH, _GW)
    buf = _sc_gather(table65.reshape(-1), git)
    r = buf.reshape(_NBLK, 8, 8, 8, 128)
    out = r.transpose(2, 4, 0, 1, 3)
    return out.reshape(gi.shape[0], _NB, _NB, _NB, _ED)
